# baseline (speedup 1.0000x reference)
import functools

import jax
import jax.numpy as jnp
from jax import lax
from jax.experimental import pallas as pl
from jax.experimental.pallas import tpu as pltpu
from jax.experimental.pallas import tpu_sc as plsc

VOCAB = 100000
N_EMBD = 768
BATCH = 4
SEQ_LEN = 2048

_LANES = 16
_NC = 2
_NS = 16
_NW = _NC * _NS

_S_PER_W = SEQ_LEN // _NW
_CHUNK = 32
_NH = _S_PER_W // _CHUNK
_NSLOT = 4
_AHEAD = 2
_ROW_SLICES = N_EMBD // _LANES

_CHUNKS = [(h, b) for h in range(_NH) for b in range(BATCH)]


def _emb_body(ids_hbm, table_hbm, pos_hbm, out_hbm,
              idx_v, tok_v, pos_v, gsem, osem, psem, isem):
    wid = lax.axis_index("s") * _NC + lax.axis_index("c")
    s_base = wid * _S_PER_W

    id_cps = []
    for b in range(BATCH):
        id_cps.append(pltpu.async_copy(
            ids_hbm.at[b, pl.ds(pl.multiple_of(s_base, _S_PER_W), _S_PER_W)],
            idx_v.at[b], isem))
    pos_cp = [pltpu.async_copy(
        pos_hbm.at[pl.ds(pl.multiple_of(s_base, _S_PER_W), _CHUNK)],
        pos_v, psem)]
    for cp in id_cps:
        cp.wait()

    def issue_gather(c, slot):
        h, b = _CHUNKS[c]
        idx = idx_v.at[b, pl.ds(h * _CHUNK, _CHUNK)]
        return pltpu.async_copy(table_hbm.at[idx], tok_v.at[slot],
                                gsem.at[slot])

    n = len(_CHUNKS)
    gather_cp = [None] * _NSLOT
    out_cp = [None] * _NSLOT
    for c in range(_AHEAD):
        gather_cp[c] = issue_gather(c, c)
    for c in range(n):
        slot = c % _NSLOT
        h, b = _CHUNKS[c]
        if c + _AHEAD < n:
            s2 = (c + _AHEAD) % _NSLOT
            if out_cp[s2] is not None:
                out_cp[s2].wait()
                out_cp[s2] = None
            gather_cp[s2] = issue_gather(c + _AHEAD, s2)
        if b == 0 and pos_cp:
            pos_cp.pop(0).wait()
        with jax.named_scope("gwait"):
            gather_cp[slot].wait()

        with jax.named_scope("addloop"):
            @plsc.parallel_loop(0, _CHUNK)
            def add_row(r):
                for j in range(_ROW_SLICES):
                    sl = pl.ds(j * _LANES, _LANES)
                    plsc.addupdate(tok_v.at[slot, r, sl], pos_v[r, sl])

        if b == BATCH - 1 and h + 1 < _NH:
            s_off = pl.multiple_of(s_base + (h + 1) * _CHUNK, _CHUNK)
            pos_cp.append(pltpu.async_copy(pos_hbm.at[pl.ds(s_off, _CHUNK)],
                                           pos_v, psem))
        s_off = pl.multiple_of(s_base + h * _CHUNK, _CHUNK)
        out_cp[slot] = pltpu.async_copy(
            tok_v.at[slot], out_hbm.at[b, pl.ds(s_off, _CHUNK)],
            osem.at[slot])
    with jax.named_scope("drain"):
        for cp in out_cp:
            if cp is not None:
                cp.wait()


@jax.jit
def _emb_call(input_ids, token_table, position_table):
    mesh = plsc.VectorSubcoreMesh(core_axis_name="c", subcore_axis_name="s")
    k = functools.partial(
        pl.kernel,
        out_type=jax.ShapeDtypeStruct((BATCH, SEQ_LEN, N_EMBD), jnp.float32),
        mesh=mesh,
        scratch_types=[
            pltpu.VMEM((BATCH, _S_PER_W), jnp.int32),
            pltpu.VMEM((_NSLOT, _CHUNK, N_EMBD), jnp.float32),
            pltpu.VMEM((_CHUNK, N_EMBD), jnp.float32),
            pltpu.SemaphoreType.DMA((_NSLOT,)),
            pltpu.SemaphoreType.DMA((_NSLOT,)),
            pltpu.SemaphoreType.DMA,
            pltpu.SemaphoreType.DMA,
        ],
    )(_emb_body)
    return k(input_ids, token_table, position_table)


def kernel(input_ids, token_table, position_table):
    return _emb_call(input_ids.astype(jnp.int32), token_table, position_table)


# speedup vs baseline: 1.0534x; 1.0534x over previous
import functools

import jax
import jax.numpy as jnp
from jax import lax
from jax.experimental import pallas as pl
from jax.experimental.pallas import tpu as pltpu
from jax.experimental.pallas import tpu_sc as plsc

VOCAB = 100000
N_EMBD = 768
BATCH = 4
SEQ_LEN = 2048

_LANES = 16
_NC = 2
_NS = 16
_NW = _NC * _NS

_S_PER_W = SEQ_LEN // _NW
_G = 16
_NG = _S_PER_W // _G
_ROW_SLICES = N_EMBD // _LANES


def _emb_body(ids_hbm, table_hbm, pos_hbm, out_hbm,
              idx_v, tok_v, pos_v, gsem, osem, psem, isem):
    wid = lax.axis_index("s") * _NC + lax.axis_index("c")
    s_base = wid * _S_PER_W

    id_cps = []
    for b in range(BATCH):
        id_cps.append(pltpu.async_copy(
            ids_hbm.at[b, pl.ds(pl.multiple_of(s_base, _S_PER_W), _S_PER_W)],
            idx_v.at[b], isem))
    pos_cp = [pltpu.async_copy(
        pos_hbm.at[pl.ds(pl.multiple_of(s_base, _S_PER_W), _G)],
        pos_v.at[0], psem.at[0])]
    for cp in id_cps:
        cp.wait()

    def issue_gathers(g, slot):
        cps = []
        for b in range(BATCH):
            idx = idx_v.at[b, pl.ds(g * _G, _G)]
            cps.append(pltpu.async_copy(table_hbm.at[idx], tok_v.at[slot, b],
                                        gsem.at[slot]))
        return cps

    gather_cp = [None, None]
    out_cp = [None, None]
    gather_cp[0] = issue_gathers(0, 0)
    for g in range(_NG):
        slot = g % 2
        nxt = 1 - slot
        if g + 1 < _NG:
            if out_cp[nxt] is not None:
                out_cp[nxt].wait()
                out_cp[nxt] = None
            gather_cp[nxt] = issue_gathers(g + 1, nxt)
            s_off = pl.multiple_of(s_base + (g + 1) * _G, _G)
            pos_cp.append(pltpu.async_copy(pos_hbm.at[pl.ds(s_off, _G)],
                                           pos_v.at[nxt], psem.at[nxt]))
        pos_cp.pop(0).wait()
        for cp in gather_cp[slot]:
            cp.wait()

        @plsc.parallel_loop(0, _G)
        def add_row(r):
            for j in range(_ROW_SLICES):
                sl = pl.ds(j * _LANES, _LANES)
                p = pos_v[slot, r, sl]
                for b in range(BATCH):
                    plsc.addupdate(tok_v.at[slot, b, r, sl], p)

        s_off = pl.multiple_of(s_base + g * _G, _G)
        out_cp[slot] = pltpu.async_copy(
            tok_v.at[slot], out_hbm.at[:, pl.ds(s_off, _G), :],
            osem.at[slot])
    for cp in out_cp:
        if cp is not None:
            cp.wait()


@jax.jit
def _emb_call(input_ids, token_table, position_table):
    mesh = plsc.VectorSubcoreMesh(core_axis_name="c", subcore_axis_name="s")
    k = functools.partial(
        pl.kernel,
        out_type=jax.ShapeDtypeStruct((BATCH, SEQ_LEN, N_EMBD), jnp.float32),
        mesh=mesh,
        scratch_types=[
            pltpu.VMEM((BATCH, _S_PER_W), jnp.int32),
            pltpu.VMEM((2, BATCH, _G, N_EMBD), jnp.float32),
            pltpu.VMEM((2, _G, N_EMBD), jnp.float32),
            pltpu.SemaphoreType.DMA((2,)),
            pltpu.SemaphoreType.DMA((2,)),
            pltpu.SemaphoreType.DMA((2,)),
            pltpu.SemaphoreType.DMA,
        ],
    )(_emb_body)
    return k(input_ids, token_table, position_table)


def kernel(input_ids, token_table, position_table):
    return _emb_call(input_ids.astype(jnp.int32), token_table, position_table)


# speedup vs baseline: 1.0552x; 1.0018x over previous
import functools

import jax
import jax.numpy as jnp
from jax import lax
from jax.experimental import pallas as pl
from jax.experimental.pallas import tpu as pltpu
from jax.experimental.pallas import tpu_sc as plsc

VOCAB = 100000
N_EMBD = 768
BATCH = 4
SEQ_LEN = 2048

_LANES = 16
_NC = 2
_NS = 16
_NW = _NC * _NS

_S_PER_W = SEQ_LEN // _NW
_G = 16
_NG = _S_PER_W // _G
_ROW_SLICES = N_EMBD // _LANES


def _emb_body(ids_hbm, table_hbm, pos_hbm, out_hbm,
              idx_v, tok_v, pos_v, gsem, osem, psem, isem):
    wid = lax.axis_index("s") * _NC + lax.axis_index("c")
    s_base = wid * _S_PER_W

    id_cps = []
    for b in range(BATCH):
        id_cps.append(pltpu.async_copy(
            ids_hbm.at[b, pl.ds(pl.multiple_of(s_base, _S_PER_W), _S_PER_W)],
            idx_v.at[b], isem))
    pos_cp = [pltpu.async_copy(
        pos_hbm.at[pl.ds(pl.multiple_of(s_base, _S_PER_W), _G)],
        pos_v.at[0], psem.at[0])]
    for cp in id_cps:
        cp.wait()

    def issue_gathers(g, slot):
        cps = []
        for b in range(BATCH):
            idx = idx_v.at[b, pl.ds(g * _G, _G)]
            cps.append(pltpu.async_copy(table_hbm.at[idx], tok_v.at[slot, b],
                                        gsem.at[slot]))
        return cps

    gather_cp = [None, None]
    out_cp = [None, None]
    gather_cp[0] = issue_gathers(0, 0)
    for g in range(_NG):
        slot = g % 2
        nxt = 1 - slot
        if g + 1 < _NG:
            if out_cp[nxt] is not None:
                out_cp[nxt].wait()
                out_cp[nxt] = None
            s_off = pl.multiple_of(s_base + (g + 1) * _G, _G)
            pos_cp.append(pltpu.async_copy(pos_hbm.at[pl.ds(s_off, _G)],
                                           pos_v.at[nxt], psem.at[nxt]))
            gather_cp[nxt] = issue_gathers(g + 1, nxt)
        pos_cp.pop(0).wait()
        for cp in gather_cp[slot]:
            cp.wait()

        @plsc.parallel_loop(0, _G)
        def add_row(r):
            for j in range(_ROW_SLICES):
                sl = pl.ds(j * _LANES, _LANES)
                p = pos_v[slot, r, sl]
                for b in range(BATCH):
                    plsc.addupdate(tok_v.at[slot, b, r, sl], p)

        s_off = pl.multiple_of(s_base + g * _G, _G)
        out_cp[slot] = pltpu.async_copy(
            tok_v.at[slot], out_hbm.at[:, pl.ds(s_off, _G), :],
            osem.at[slot])
    for cp in out_cp:
        if cp is not None:
            cp.wait()


@jax.jit
def _emb_call(input_ids, token_table, position_table):
    mesh = plsc.VectorSubcoreMesh(core_axis_name="c", subcore_axis_name="s")
    k = functools.partial(
        pl.kernel,
        out_type=jax.ShapeDtypeStruct((BATCH, SEQ_LEN, N_EMBD), jnp.float32),
        mesh=mesh,
        scratch_types=[
            pltpu.VMEM((BATCH, _S_PER_W), jnp.int32),
            pltpu.VMEM((2, BATCH, _G, N_EMBD), jnp.float32),
            pltpu.VMEM((2, _G, N_EMBD), jnp.float32),
            pltpu.SemaphoreType.DMA((2,)),
            pltpu.SemaphoreType.DMA((2,)),
            pltpu.SemaphoreType.DMA((2,)),
            pltpu.SemaphoreType.DMA,
        ],
    )(_emb_body)
    return k(input_ids, token_table, position_table)


def kernel(input_ids, token_table, position_table):
    return _emb_call(input_ids.astype(jnp.int32), token_table, position_table)
